# baseline (speedup 1.0000x reference)
import functools

import jax
import jax.numpy as jnp
from jax import lax
from jax.experimental import pallas as pl
from jax.experimental.pallas import tpu as pltpu
from jax.experimental.pallas import tpu_sc as plsc

R, C = 16384, 512
OC = C // 2
NW = 32
IN_PER_W = R * C // NW
OUT_PER_W = IN_PER_W // 2
N_CHUNK = 8
CHUNK_IN = IN_PER_W // N_CHUNK
CHUNK_OUT = CHUNK_IN // 2
LANES = 16

_mesh = plsc.VectorSubcoreMesh(core_axis_name="c", subcore_axis_name="s")


@functools.partial(
    pl.kernel,
    mesh=_mesh,
    out_type=jax.ShapeDtypeStruct((R * OC,), jnp.float32),
    scratch_types=[
        pltpu.VMEM((CHUNK_IN,), jnp.float32),
        pltpu.VMEM((CHUNK_OUT,), jnp.float32),
        pltpu.SemaphoreType.DMA,
    ],
    compiler_params=pltpu.CompilerParams(needs_layout_passes=False),
)
def _deinterleave(in_hbm, out_hbm, in_v, out_v, sem):
    wid = lax.axis_index("s") * 2 + lax.axis_index("c")
    in_base = wid * IN_PER_W
    out_base = wid * OUT_PER_W
    iota2 = lax.iota(jnp.int32, LANES) * 2

    def chunk(c):
        pltpu.sync_copy(in_hbm.at[pl.ds(in_base + c * CHUNK_IN, CHUNK_IN)],
                        in_v)

        def body(i, _):
            idx = iota2 + i * (2 * LANES)
            out_v[pl.ds(i * LANES, LANES)] = plsc.load_gather(in_v, [idx])
            return 0

        lax.fori_loop(0, CHUNK_OUT // LANES, body, 0)
        pltpu.sync_copy(out_v,
                        out_hbm.at[pl.ds(out_base + c * CHUNK_OUT, CHUNK_OUT)])

    for c in range(N_CHUNK):
        chunk(c)


def kernel(inputs):
    flat = inputs.reshape(-1)
    out = _deinterleave(flat)
    return out.reshape(R, OC)


# speedup vs baseline: 1.4352x; 1.4352x over previous
import functools

import jax
import jax.numpy as jnp
from jax import lax
from jax.experimental import pallas as pl
from jax.experimental.pallas import tpu as pltpu
from jax.experimental.pallas import tpu_sc as plsc

R, C = 16384, 512
OC = C // 2
NW = 32
IN_PER_W = R * C // NW
OUT_PER_W = IN_PER_W // 2
N_CHUNK = 8
CHUNK_IN = IN_PER_W // N_CHUNK
CHUNK_OUT = CHUNK_IN // 2
LANES = 16

_mesh = plsc.VectorSubcoreMesh(core_axis_name="c", subcore_axis_name="s")


@functools.partial(
    pl.kernel,
    mesh=_mesh,
    out_type=jax.ShapeDtypeStruct((R * OC,), jnp.float32),
    scratch_types=[
        pltpu.VMEM((CHUNK_IN,), jnp.float32),
        pltpu.VMEM((CHUNK_IN,), jnp.float32),
        pltpu.VMEM((CHUNK_OUT,), jnp.float32),
        pltpu.VMEM((CHUNK_OUT,), jnp.float32),
        pltpu.SemaphoreType.DMA,
        pltpu.SemaphoreType.DMA,
    ],
    compiler_params=pltpu.CompilerParams(needs_layout_passes=False),
)
def _deinterleave(in_hbm, out_hbm, in_v0, in_v1, out_v0, out_v1,
                  in_sem, out_sem):
    wid = lax.axis_index("s") * 2 + lax.axis_index("c")
    in_base = wid * IN_PER_W
    out_base = wid * OUT_PER_W
    iota2 = lax.iota(jnp.int32, LANES) * 2
    in_bufs = (in_v0, in_v1)
    out_bufs = (out_v0, out_v1)

    def in_copy(c):
        return pltpu.async_copy(
            in_hbm.at[pl.ds(in_base + c * CHUNK_IN, CHUNK_IN)],
            in_bufs[c % 2], in_sem)

    def out_copy(c):
        return pltpu.async_copy(
            out_bufs[c % 2],
            out_hbm.at[pl.ds(out_base + c * CHUNK_OUT, CHUNK_OUT)],
            out_sem)

    in_h = in_copy(0)
    out_h = [None, None]
    for c in range(N_CHUNK):
        in_h.wait()
        if c + 1 < N_CHUNK:
            in_h = in_copy(c + 1)
        if out_h[c % 2] is not None:
            out_h[c % 2].wait()
        iv = in_bufs[c % 2]
        ov = out_bufs[c % 2]

        @plsc.parallel_loop(0, CHUNK_OUT // LANES, 1, unroll=8)
        def _(i):
            idx = iota2 + i * (2 * LANES)
            ov[pl.ds(i * LANES, LANES)] = plsc.load_gather(iv, [idx])

        out_h[c % 2] = out_copy(c)
    out_h[0].wait()
    out_h[1].wait()


def kernel(inputs):
    flat = inputs.reshape(-1)
    out = _deinterleave(flat)
    return out.reshape(R, OC)


# speedup vs baseline: 3.0253x; 2.1080x over previous
import functools

import jax
import jax.numpy as jnp
from jax import lax
from jax.experimental import pallas as pl
from jax.experimental.pallas import tpu as pltpu
from jax.experimental.pallas import tpu_sc as plsc

R, C = 16384, 512
OC = C // 2
NW = 32
ROWS_PER_W = R // NW
N_CHUNK = 8
CH_ROWS = ROWS_PER_W // N_CHUNK
LANES = 16
VECS_PER_ROW = OC // LANES

_mesh = plsc.VectorSubcoreMesh(core_axis_name="c", subcore_axis_name="s")


@functools.partial(
    pl.kernel,
    mesh=_mesh,
    out_type=jax.ShapeDtypeStruct((R, OC), jnp.float32),
    scratch_types=[
        pltpu.VMEM((CH_ROWS, C), jnp.float32),
        pltpu.VMEM((CH_ROWS, C), jnp.float32),
        pltpu.VMEM((CH_ROWS, OC), jnp.float32),
        pltpu.VMEM((CH_ROWS, OC), jnp.float32),
        pltpu.SemaphoreType.DMA,
        pltpu.SemaphoreType.DMA,
    ],
    compiler_params=pltpu.CompilerParams(needs_layout_passes=False),
)
def _deinterleave(in_hbm, out_hbm, in_v0, in_v1, out_v0, out_v1,
                  in_sem, out_sem):
    wid = lax.axis_index("s") * 2 + lax.axis_index("c")
    row_base = wid * ROWS_PER_W
    iota2 = lax.iota(jnp.int32, LANES) * 2
    in_bufs = (in_v0, in_v1)
    out_bufs = (out_v0, out_v1)

    def in_copy(c):
        return pltpu.async_copy(
            in_hbm.at[pl.ds(row_base + c * CH_ROWS, CH_ROWS), :],
            in_bufs[c % 2], in_sem)

    def out_copy(c):
        return pltpu.async_copy(
            out_bufs[c % 2],
            out_hbm.at[pl.ds(row_base + c * CH_ROWS, CH_ROWS), :],
            out_sem)

    in_h = in_copy(0)
    out_h = [None, None]
    for c in range(N_CHUNK):
        in_h.wait()
        if c + 1 < N_CHUNK:
            in_h = in_copy(c + 1)
        if out_h[c % 2] is not None:
            out_h[c % 2].wait()
        iv = in_bufs[c % 2]
        ov = out_bufs[c % 2]

        @plsc.parallel_loop(0, CH_ROWS * VECS_PER_ROW, 1, unroll=8)
        def _(i):
            r = i >> 4
            j = i & (VECS_PER_ROW - 1)
            col = iota2 + j * (2 * LANES)
            row = jnp.full((LANES,), r, jnp.int32)
            ov[r, pl.ds(j * LANES, LANES)] = plsc.load_gather(iv, [row, col])

        out_h[c % 2] = out_copy(c)
    out_h[0].wait()
    out_h[1].wait()


def kernel(inputs):
    return _deinterleave(inputs)
